# baseline (speedup 1.0000x reference)
import functools

import jax
import jax.numpy as jnp
import numpy as np
from jax import lax
from jax.experimental import pallas as pl
from jax.experimental.pallas import tpu as pltpu
from jax.experimental.pallas import tpu_sc as plsc

F32 = jnp.float32
I32 = jnp.int32


def _sc_mesh():
    return plsc.VectorSubcoreMesh(core_axis_name="c", subcore_axis_name="s",
                                  num_cores=2, num_subcores=16)


def _per_tile_row_copy(s, n, copy_fn):
    rpt = (n // 16) // 8 * 8
    last = n - 15 * rpt

    @pl.when(s < 15)
    def _():
        copy_fn(pl.multiple_of(s * rpt, 8), rpt)

    @pl.when(s == 15)
    def _():
        copy_fn(15 * rpt, last)


def _deg_hist(dst, zeros_img, ones_img, n, e):
    ch = 128
    e_core = e // 2
    e_sub = e_core // 16
    nch = e_sub // ch
    tail = e_sub - nch * ch

    @functools.partial(
        pl.kernel,
        out_type=(jax.ShapeDtypeStruct((n, 128), F32),
                  jax.ShapeDtypeStruct((n, 128), F32)),
        mesh=_sc_mesh(),
        scratch_types=[
            pltpu.VMEM_SHARED((n, 128), F32),
            pltpu.VMEM((ch,), I32), pltpu.VMEM((ch,), I32),
            pltpu.VMEM((ch,), I32), pltpu.VMEM((ch,), I32),
            pltpu.VMEM((ch, 128), F32),
            pltpu.VMEM((max(tail, 1),), I32),
            pltpu.SemaphoreType.DMA, pltpu.SemaphoreType.DMA,
            pltpu.SemaphoreType.DMA, pltpu.SemaphoreType.DMA,
        ],
    )
    def k(dst_h, zeros_h, ones_h, out0, out1, acc, di0, di1, di2, di3,
          ones_v, idxt_v, semI0, semI1, semI2, semI3):
        c = lax.axis_index("c")
        s = lax.axis_index("s")
        eb = c * e_core + s * e_sub
        islots = ((di0, semI0), (di1, semI1), (di2, semI2), (di3, semI3))

        def idx_load(j, sl):
            b = pl.multiple_of(eb + j * ch, 8)
            pltpu.async_copy(dst_h.at[pl.ds(b, ch)], sl[0], sl[1])

        def idx_wait(sl):
            pltpu.make_async_copy(dst_h.at[pl.ds(0, ch)], sl[0], sl[1]).wait()

        def scat(sl):
            pltpu.sync_copy(ones_v, acc.at[sl[0]], add=True)

        idx_load(0, islots[0])
        idx_load(1, islots[1])
        _per_tile_row_copy(s, n, lambda r0, nr: pltpu.sync_copy(
            zeros_h.at[pl.ds(r0, nr)], acc.at[pl.ds(r0, nr)]))
        pltpu.sync_copy(ones_h, ones_v)
        plsc.subcore_barrier()

        def segment(jb, sA, sB, sC, sD):
            @pl.when(jb + 2 < nch)
            def _():
                idx_load(jb + 2, sC)

            idx_wait(sA)
            scat(sA)

            @pl.when(jb + 3 < nch)
            def _():
                idx_load(jb + 3, sD)

            idx_wait(sB)
            scat(sB)

        def quad(m, carry):
            j0 = 4 * m
            segment(j0, islots[0], islots[1], islots[2], islots[3])
            segment(j0 + 2, islots[2], islots[3], islots[0], islots[1])
            return carry

        lax.fori_loop(0, nch // 4, quad, 0)
        rem = nch % 4
        jr = nch - rem
        if rem >= 2:
            segment(jr, islots[jr % 4], islots[(jr + 1) % 4],
                    islots[(jr + 2) % 4], islots[(jr + 3) % 4])
        if rem % 2:
            sl = islots[(nch - 1) % 4]
            idx_wait(sl)
            scat(sl)
        if tail:
            b = pl.multiple_of(eb + nch * ch, 8)
            pltpu.sync_copy(dst_h.at[pl.ds(b, tail)], idxt_v)
            pltpu.sync_copy(ones_v.at[pl.ds(0, tail)], acc.at[idxt_v], add=True)
        plsc.subcore_barrier()

        @pl.when(c == 0)
        def _():
            _per_tile_row_copy(s, n, lambda r0, nr: pltpu.sync_copy(
                acc.at[pl.ds(r0, nr)], out0.at[pl.ds(r0, nr)]))

        @pl.when(c == 1)
        def _():
            _per_tile_row_copy(s, n, lambda r0, nr: pltpu.sync_copy(
                acc.at[pl.ds(r0, nr)], out1.at[pl.ds(r0, nr)]))

    return k(dst, zeros_img, ones_img)


def _sc_aggregate(ya, yb, src, dst, n, e):
    ch = 128
    e_sub = e // 16
    nch = e_sub // ch
    tail = e_sub - nch * ch
    assert nch % 2 == 0

    @functools.partial(
        pl.kernel,
        out_type=(jax.ShapeDtypeStruct((n, 128), F32),
                  jax.ShapeDtypeStruct((n, 128), F32)),
        mesh=_sc_mesh(),
        scratch_types=[
            pltpu.VMEM_SHARED((n, 128), F32),
            pltpu.VMEM((ch,), I32), pltpu.VMEM((ch,), I32),
            pltpu.VMEM((ch,), I32), pltpu.VMEM((ch,), I32),
            pltpu.VMEM((ch,), I32), pltpu.VMEM((ch,), I32),
            pltpu.VMEM((ch,), I32), pltpu.VMEM((ch,), I32),
            pltpu.VMEM((ch, 128), F32),
            pltpu.VMEM((ch, 128), F32),
            pltpu.VMEM((max(tail, 1),), I32),
            pltpu.VMEM((max(tail, 1),), I32),
            pltpu.VMEM((max(tail, 1), 128), F32),
            pltpu.SemaphoreType.DMA, pltpu.SemaphoreType.DMA,
            pltpu.SemaphoreType.DMA, pltpu.SemaphoreType.DMA,
            pltpu.SemaphoreType.DMA, pltpu.SemaphoreType.DMA,
        ],
    )
    def k(ya_h, yb_h, src_h, dst_h, za_h, zb_h,
          acc, si0, si1, si2, si3, di0, di1, di2, di3, rows0, rows1,
          sit, dit, rowst, semA, semB, semI0, semI1, semI2, semI3):
        c = lax.axis_index("c")
        s = lax.axis_index("s")
        eb = s * e_sub

        def run(y_h, z_h):
            islots = ((si0, di0, semI0), (si1, di1, semI1),
                      (si2, di2, semI2), (si3, di3, semI3))

            def idx_load(j, sl):
                si, di, semI = sl
                b = pl.multiple_of(eb + j * ch, 8)
                pltpu.async_copy(src_h.at[pl.ds(b, ch)], si, semI)
                pltpu.async_copy(dst_h.at[pl.ds(b, ch)], di, semI)

            def idx_wait(sl):
                si, di, semI = sl
                pltpu.make_async_copy(src_h.at[pl.ds(0, ch)], si, semI).wait()
                pltpu.make_async_copy(dst_h.at[pl.ds(0, ch)], di, semI).wait()

            def gather(sl, buf, sem):
                pltpu.async_copy(y_h.at[sl[0]], buf, sem)

            def gather_wait(sl, buf, sem):
                pltpu.make_async_copy(y_h.at[sl[0]], buf, sem).wait()

            def scat(sl, buf):
                pltpu.sync_copy(buf, acc.at[sl[1]], add=True)

            idx_load(0, islots[0])
            idx_load(1, islots[1])
            _per_tile_row_copy(s, n, lambda r0, nr: pltpu.sync_copy(
                y_h.at[pl.ds(r0, nr)], acc.at[pl.ds(r0, nr)]))
            plsc.subcore_barrier()

            idx_wait(islots[0])
            gather(islots[0], rows0, semA)

            def segment(jb, sA, sB, sC, sD):
                idx_wait(sB)
                gather(sB, rows1, semB)
                gather_wait(sA, rows0, semA)

                @pl.when(jb + 2 < nch)
                def _():
                    idx_load(jb + 2, sC)

                scat(sA, rows0)

                @pl.when(jb + 2 < nch)
                def _():
                    idx_wait(sC)
                    gather(sC, rows0, semA)

                gather_wait(sB, rows1, semB)
                scat(sB, rows1)

                @pl.when(jb + 3 < nch)
                def _():
                    idx_load(jb + 3, sD)

            def quad(m, carry):
                j0 = 4 * m
                segment(j0, islots[0], islots[1], islots[2], islots[3])
                segment(j0 + 2, islots[2], islots[3], islots[0], islots[1])
                return carry

            lax.fori_loop(0, nch // 4, quad, 0)
            if nch % 4:
                assert nch % 4 == 2
                segment(nch - 2, islots[0], islots[1], islots[2], islots[3])
            if tail:
                b = pl.multiple_of(eb + nch * ch, 8)
                pltpu.sync_copy(src_h.at[pl.ds(b, tail)], sit)
                pltpu.sync_copy(dst_h.at[pl.ds(b, tail)], dit)
                pltpu.async_copy(y_h.at[sit], rowst, semA).wait()
                pltpu.sync_copy(rowst, acc.at[dit], add=True)
            plsc.subcore_barrier()
            _per_tile_row_copy(s, n, lambda r0, nr: pltpu.sync_copy(
                acc.at[pl.ds(r0, nr)], z_h.at[pl.ds(r0, nr)]))

        @pl.when(c == 0)
        def _():
            run(ya_h, za_h)

        @pl.when(c == 1)
        def _():
            run(yb_h, zb_h)

    return k(ya, yb, src, dst)


def _dinv_block(d0, d1):
    return lax.rsqrt(d0[:, 0:1] + d1[:, 0:1] + 1.0)


def _dot16(a, w):
    return jnp.dot(a.astype(jnp.bfloat16), w.astype(jnp.bfloat16),
                   preferred_element_type=F32)


def _tc_prescale(x, deg0, deg1, n):
    B = 1000

    def body(x_r, d0_r, d1_r, ya_r, yb_r):
        dv = _dinv_block(d0_r, d1_r)
        xv = x_r[...]
        ya_r[...] = xv[:, :128] * dv
        yb_r[...] = xv[:, 128:] * dv

    return pl.pallas_call(
        body,
        grid=(n // B,),
        in_specs=[
            pl.BlockSpec((B, 256), lambda i: (i, 0)),
            pl.BlockSpec((B, 128), lambda i: (i, 0)),
            pl.BlockSpec((B, 128), lambda i: (i, 0)),
        ],
        out_specs=[pl.BlockSpec((B, 128), lambda i: (i, 0))] * 2,
        out_shape=[jax.ShapeDtypeStruct((n, 128), F32)] * 2,
    )(x, deg0, deg1)


def _tc_conv_mid(z1a, z1b, deg0, deg1, W1, b1, W2, n):
    B = 1000

    def body(za_r, zb_r, d0_r, d1_r, W1_r, b1_r, W2_r, ya_r, yb_r):
        dv = _dinv_block(d0_r, d1_r)
        s1 = jnp.concatenate([za_r[...], zb_r[...]], axis=1) * dv
        h1 = jnp.tanh(_dot16(s1, W1_r[...])
                      + b1_r[...])
        t = _dot16(h1, W2_r[...]) * dv
        ya_r[...] = t[:, :128]
        yb_r[...] = t[:, 128:]

    return pl.pallas_call(
        body,
        grid=(n // B,),
        in_specs=[
            pl.BlockSpec((B, 128), lambda i: (i, 0)),
            pl.BlockSpec((B, 128), lambda i: (i, 0)),
            pl.BlockSpec((B, 128), lambda i: (i, 0)),
            pl.BlockSpec((B, 128), lambda i: (i, 0)),
            pl.BlockSpec((256, 512), lambda i: (0, 0)),
            pl.BlockSpec((1, 512), lambda i: (0, 0)),
            pl.BlockSpec((512, 256), lambda i: (0, 0)),
        ],
        out_specs=[pl.BlockSpec((B, 128), lambda i: (i, 0))] * 2,
        out_shape=[jax.ShapeDtypeStruct((n, 128), F32)] * 2,
    )(z1a, z1b, deg0, deg1, W1, b1.reshape(1, 512), W2)


def _tc_heads(z2a, z2b, deg0, deg1, b2, Wc1, bc1, Wc2, bc2, W3p, b3p,
              Wr1, br1, Wr2, br2, Wr3, br3, n):
    B = 1000

    def body(za_r, zb_r, d0_r, d1_r, b2_r, Wc1_r, bc1_r, Wc2_r, bc2_r,
             W3_r, b3_r, Wr1_r, br1_r, Wr2_r, br2_r, Wr3_r, br3_r,
             c_r, r_r, h_r):
        dv = _dinv_block(d0_r, d1_r)
        z = jnp.concatenate([za_r[...], zb_r[...]], axis=1) * dv
        h = jnp.tanh(z + b2_r[...])
        h_r[...] = h
        c1 = jnp.maximum(_dot16(h, Wc1_r[...])
                         + bc1_r[...], 0.0)
        c2 = jnp.maximum(_dot16(c1, Wc2_r[...])
                         + bc2_r[...], 0.0)
        logits = _dot16(c2, W3_r[...]) + b3_r[...]
        m = jnp.max(logits, axis=1, keepdims=True)
        ex = jnp.exp(logits - m)
        p = ex / jnp.sum(ex, axis=1, keepdims=True)
        c_r[...] = p[:, :16]
        r1 = jnp.maximum(_dot16(h, Wr1_r[...])
                         + br1_r[...], 0.0)
        r2 = jnp.maximum(_dot16(r1, Wr2_r[...])
                         + br2_r[...], 0.0)
        r_r[...] = _dot16(r2, Wr3_r[...]) + br3_r[...]

    full = lambda a, b: pl.BlockSpec((a, b), lambda i: (0, 0))
    return pl.pallas_call(
        body,
        grid=(n // B,),
        in_specs=[
            pl.BlockSpec((B, 128), lambda i: (i, 0)),
            pl.BlockSpec((B, 128), lambda i: (i, 0)),
            pl.BlockSpec((B, 128), lambda i: (i, 0)),
            pl.BlockSpec((B, 128), lambda i: (i, 0)),
            full(1, 256), full(256, 256), full(1, 256), full(256, 128),
            full(1, 128), full(128, 128), full(1, 128), full(256, 256),
            full(1, 256), full(256, 512), full(1, 512), full(512, 256),
            full(1, 256),
        ],
        out_specs=[
            pl.BlockSpec((B, 16), lambda i: (i, 0)),
            pl.BlockSpec((B, 256), lambda i: (i, 0)),
            pl.BlockSpec((B, 256), lambda i: (i, 0)),
        ],
        out_shape=[
            jax.ShapeDtypeStruct((n, 16), F32),
            jax.ShapeDtypeStruct((n, 256), F32),
            jax.ShapeDtypeStruct((n, 256), F32),
        ],
    )(z2a, z2b, deg0, deg1, b2, Wc1, bc1, Wc2, bc2, W3p, b3p,
      Wr1, br1, Wr2, br2, Wr3, br3)


def kernel(x, edge_index, W_gc1, b_gc1, W_gc2, b_gc2, W_c1, b_c1, g_c1, be_c1,
           W_c2, b_c2, g_c2, be_c2, W_c3, b_c3, W_r1, b_r1, W_r2, b_r2,
           W_r3, b_r3):
    n, d = x.shape
    e = edge_index.shape[1]
    src = edge_index[0]
    dst = edge_index[1]

    zeros_img = jnp.zeros((n, 128), F32)
    ones_img = jnp.ones((128, 128), F32)

    bn = np.float32(1.0 / np.sqrt(1.0 + 1e-5))
    Wc1 = W_c1 * (g_c1 * bn)[None, :]
    bc1 = b_c1 * (g_c1 * bn) + be_c1
    Wc2 = W_c2 * (g_c2 * bn)[None, :]
    bc2 = b_c2 * (g_c2 * bn) + be_c2
    W3p = jnp.pad(W_c3, ((0, 0), (0, 112)))
    b3p = jnp.pad(b_c3, (0, 112), constant_values=-1e30)

    deg0, deg1 = _deg_hist(dst, zeros_img, ones_img, n, e)
    y1a, y1b = _tc_prescale(x, deg0, deg1, n)
    z1a, z1b = _sc_aggregate(y1a, y1b, src, dst, n, e)
    y2a, y2b = _tc_conv_mid(z1a, z1b, deg0, deg1, W_gc1, b_gc1, W_gc2, n)
    z2a, z2b = _sc_aggregate(y2a, y2b, src, dst, n, e)
    c, r, h = _tc_heads(z2a, z2b, deg0, deg1, b_gc2.reshape(1, 256),
                        Wc1, bc1.reshape(1, 256), Wc2, bc2.reshape(1, 128),
                        W3p, b3p.reshape(1, 128),
                        W_r1, b_r1.reshape(1, 256), W_r2, b_r2.reshape(1, 512),
                        W_r3, b_r3.reshape(1, 256), n)
    return (c, r, h)


# speedup vs baseline: 1.0181x; 1.0181x over previous
import functools

import jax
import jax.numpy as jnp
import numpy as np
from jax import lax
from jax.experimental import pallas as pl
from jax.experimental.pallas import tpu as pltpu
from jax.experimental.pallas import tpu_sc as plsc

F32 = jnp.float32
I32 = jnp.int32


def _sc_mesh():
    return plsc.VectorSubcoreMesh(core_axis_name="c", subcore_axis_name="s",
                                  num_cores=2, num_subcores=16)


def _per_tile_row_copy(s, n, copy_fn):
    rpt = (n // 16) // 8 * 8
    last = n - 15 * rpt

    @pl.when(s < 15)
    def _():
        copy_fn(pl.multiple_of(s * rpt, 8), rpt)

    @pl.when(s == 15)
    def _():
        copy_fn(15 * rpt, last)


def _deg_hist(dst, zeros_img, ones_img, n, e):
    ch = 128
    e_core = e // 2
    e_sub = e_core // 16
    nch = e_sub // ch
    tail = e_sub - nch * ch

    @functools.partial(
        pl.kernel,
        out_type=(jax.ShapeDtypeStruct((n, 128), F32),
                  jax.ShapeDtypeStruct((n, 128), F32)),
        mesh=_sc_mesh(),
        scratch_types=[
            pltpu.VMEM_SHARED((n, 128), F32),
            pltpu.VMEM((ch,), I32), pltpu.VMEM((ch,), I32),
            pltpu.VMEM((ch,), I32), pltpu.VMEM((ch,), I32),
            pltpu.VMEM((ch, 128), F32),
            pltpu.VMEM((max(tail, 1),), I32),
            pltpu.SemaphoreType.DMA, pltpu.SemaphoreType.DMA,
            pltpu.SemaphoreType.DMA, pltpu.SemaphoreType.DMA,
        ],
    )
    def k(dst_h, zeros_h, ones_h, out0, out1, acc, di0, di1, di2, di3,
          ones_v, idxt_v, semI0, semI1, semI2, semI3):
        c = lax.axis_index("c")
        s = lax.axis_index("s")
        eb = c * e_core + s * e_sub
        islots = ((di0, semI0), (di1, semI1), (di2, semI2), (di3, semI3))

        def idx_load(j, sl):
            b = pl.multiple_of(eb + j * ch, 8)
            pltpu.async_copy(dst_h.at[pl.ds(b, ch)], sl[0], sl[1])

        def idx_wait(sl):
            pltpu.make_async_copy(dst_h.at[pl.ds(0, ch)], sl[0], sl[1]).wait()

        def scat(sl):
            pltpu.sync_copy(ones_v, acc.at[sl[0]], add=True)

        idx_load(0, islots[0])
        idx_load(1, islots[1])
        _per_tile_row_copy(s, n, lambda r0, nr: pltpu.sync_copy(
            zeros_h.at[pl.ds(r0, nr)], acc.at[pl.ds(r0, nr)]))
        pltpu.sync_copy(ones_h, ones_v)
        plsc.subcore_barrier()

        def segment(jb, sA, sB, sC, sD):
            @pl.when(jb + 2 < nch)
            def _():
                idx_load(jb + 2, sC)

            idx_wait(sA)
            scat(sA)

            @pl.when(jb + 3 < nch)
            def _():
                idx_load(jb + 3, sD)

            idx_wait(sB)
            scat(sB)

        def quad(m, carry):
            j0 = 4 * m
            segment(j0, islots[0], islots[1], islots[2], islots[3])
            segment(j0 + 2, islots[2], islots[3], islots[0], islots[1])
            return carry

        lax.fori_loop(0, nch // 4, quad, 0)
        rem = nch % 4
        jr = nch - rem
        if rem >= 2:
            segment(jr, islots[jr % 4], islots[(jr + 1) % 4],
                    islots[(jr + 2) % 4], islots[(jr + 3) % 4])
        if rem % 2:
            sl = islots[(nch - 1) % 4]
            idx_wait(sl)
            scat(sl)
        if tail:
            b = pl.multiple_of(eb + nch * ch, 8)
            pltpu.sync_copy(dst_h.at[pl.ds(b, tail)], idxt_v)
            pltpu.sync_copy(ones_v.at[pl.ds(0, tail)], acc.at[idxt_v], add=True)
        plsc.subcore_barrier()

        @pl.when(c == 0)
        def _():
            _per_tile_row_copy(s, n, lambda r0, nr: pltpu.sync_copy(
                acc.at[pl.ds(r0, nr)], out0.at[pl.ds(r0, nr)]))

        @pl.when(c == 1)
        def _():
            _per_tile_row_copy(s, n, lambda r0, nr: pltpu.sync_copy(
                acc.at[pl.ds(r0, nr)], out1.at[pl.ds(r0, nr)]))

    return k(dst, zeros_img, ones_img)


def _sc_aggregate(ya, yb, src, dst, n, e):
    ch = 128
    e_sub = e // 16
    nch = e_sub // ch
    tail = e_sub - nch * ch
    assert nch % 2 == 0

    @functools.partial(
        pl.kernel,
        out_type=(jax.ShapeDtypeStruct((n, 128), F32),
                  jax.ShapeDtypeStruct((n, 128), F32)),
        mesh=_sc_mesh(),
        scratch_types=[
            pltpu.VMEM_SHARED((n, 128), F32),
            pltpu.VMEM((ch,), I32), pltpu.VMEM((ch,), I32),
            pltpu.VMEM((ch,), I32), pltpu.VMEM((ch,), I32),
            pltpu.VMEM((ch,), I32), pltpu.VMEM((ch,), I32),
            pltpu.VMEM((ch,), I32), pltpu.VMEM((ch,), I32),
            pltpu.VMEM((ch, 128), F32),
            pltpu.VMEM((ch, 128), F32),
            pltpu.VMEM((max(tail, 1),), I32),
            pltpu.VMEM((max(tail, 1),), I32),
            pltpu.VMEM((max(tail, 1), 128), F32),
            pltpu.SemaphoreType.DMA, pltpu.SemaphoreType.DMA,
            pltpu.SemaphoreType.DMA, pltpu.SemaphoreType.DMA,
            pltpu.SemaphoreType.DMA, pltpu.SemaphoreType.DMA,
        ],
    )
    def k(ya_h, yb_h, src_h, dst_h, za_h, zb_h,
          acc, si0, si1, si2, si3, di0, di1, di2, di3, rows0, rows1,
          sit, dit, rowst, semA, semB, semI0, semI1, semI2, semI3):
        c = lax.axis_index("c")
        s = lax.axis_index("s")
        eb = s * e_sub

        def run(y_h, z_h):
            islots = ((si0, di0, semI0), (si1, di1, semI1),
                      (si2, di2, semI2), (si3, di3, semI3))

            def idx_load(j, sl):
                si, di, semI = sl
                b = pl.multiple_of(eb + j * ch, 8)
                pltpu.async_copy(src_h.at[pl.ds(b, ch)], si, semI)
                pltpu.async_copy(dst_h.at[pl.ds(b, ch)], di, semI)

            def idx_wait(sl):
                si, di, semI = sl
                pltpu.make_async_copy(src_h.at[pl.ds(0, ch)], si, semI).wait()
                pltpu.make_async_copy(dst_h.at[pl.ds(0, ch)], di, semI).wait()

            def gather(sl, buf, sem):
                pltpu.async_copy(y_h.at[sl[0]], buf, sem)

            def gather_wait(sl, buf, sem):
                pltpu.make_async_copy(y_h.at[sl[0]], buf, sem).wait()

            def scat(sl, buf):
                pltpu.sync_copy(buf, acc.at[sl[1]], add=True)

            idx_load(0, islots[0])
            idx_load(1, islots[1])
            _per_tile_row_copy(s, n, lambda r0, nr: pltpu.sync_copy(
                y_h.at[pl.ds(r0, nr)], acc.at[pl.ds(r0, nr)]))
            plsc.subcore_barrier()

            idx_wait(islots[0])
            gather(islots[0], rows0, semA)

            def segment(jb, sA, sB, sC, sD):
                idx_wait(sB)
                gather(sB, rows1, semB)
                gather_wait(sA, rows0, semA)

                @pl.when(jb + 2 < nch)
                def _():
                    idx_load(jb + 2, sC)

                scat(sA, rows0)

                @pl.when(jb + 2 < nch)
                def _():
                    idx_wait(sC)
                    gather(sC, rows0, semA)

                gather_wait(sB, rows1, semB)
                scat(sB, rows1)

                @pl.when(jb + 3 < nch)
                def _():
                    idx_load(jb + 3, sD)

            def quad(m, carry):
                j0 = 4 * m
                segment(j0, islots[0], islots[1], islots[2], islots[3])
                segment(j0 + 2, islots[2], islots[3], islots[0], islots[1])
                return carry

            lax.fori_loop(0, nch // 4, quad, 0)
            if nch % 4:
                assert nch % 4 == 2
                segment(nch - 2, islots[0], islots[1], islots[2], islots[3])
            if tail:
                b = pl.multiple_of(eb + nch * ch, 8)
                pltpu.sync_copy(src_h.at[pl.ds(b, tail)], sit)
                pltpu.sync_copy(dst_h.at[pl.ds(b, tail)], dit)
                pltpu.async_copy(y_h.at[sit], rowst, semA).wait()
                pltpu.sync_copy(rowst, acc.at[dit], add=True)
            plsc.subcore_barrier()
            _per_tile_row_copy(s, n, lambda r0, nr: pltpu.sync_copy(
                acc.at[pl.ds(r0, nr)], z_h.at[pl.ds(r0, nr)]))

        @pl.when(c == 0)
        def _():
            run(ya_h, za_h)

        @pl.when(c == 1)
        def _():
            run(yb_h, zb_h)

    return k(ya, yb, src, dst)


def _dinv_block(d0, d1):
    return lax.rsqrt(d0[:, 0:1] + d1[:, 0:1] + 1.0)


def _dot16(a, w):
    return jnp.dot(a.astype(jnp.bfloat16), w.astype(jnp.bfloat16),
                   preferred_element_type=F32)


def _tc_prescale(x, deg0, deg1, n):
    B = 2000

    def body(x_r, d0_r, d1_r, ya_r, yb_r):
        dv = _dinv_block(d0_r, d1_r)
        xv = x_r[...]
        ya_r[...] = xv[:, :128] * dv
        yb_r[...] = xv[:, 128:] * dv

    return pl.pallas_call(
        body,
        grid=(n // B,),
        in_specs=[
            pl.BlockSpec((B, 256), lambda i: (i, 0)),
            pl.BlockSpec((B, 128), lambda i: (i, 0)),
            pl.BlockSpec((B, 128), lambda i: (i, 0)),
        ],
        out_specs=[pl.BlockSpec((B, 128), lambda i: (i, 0))] * 2,
        out_shape=[jax.ShapeDtypeStruct((n, 128), F32)] * 2,
    )(x, deg0, deg1)


def _tc_conv_mid(z1a, z1b, deg0, deg1, W1, b1, W2, n):
    B = 2000

    def body(za_r, zb_r, d0_r, d1_r, W1_r, b1_r, W2_r, ya_r, yb_r):
        dv = _dinv_block(d0_r, d1_r)
        s1 = jnp.concatenate([za_r[...], zb_r[...]], axis=1) * dv
        h1 = jnp.tanh(_dot16(s1, W1_r[...])
                      + b1_r[...])
        t = _dot16(h1, W2_r[...]) * dv
        ya_r[...] = t[:, :128]
        yb_r[...] = t[:, 128:]

    return pl.pallas_call(
        body,
        grid=(n // B,),
        in_specs=[
            pl.BlockSpec((B, 128), lambda i: (i, 0)),
            pl.BlockSpec((B, 128), lambda i: (i, 0)),
            pl.BlockSpec((B, 128), lambda i: (i, 0)),
            pl.BlockSpec((B, 128), lambda i: (i, 0)),
            pl.BlockSpec((256, 512), lambda i: (0, 0)),
            pl.BlockSpec((1, 512), lambda i: (0, 0)),
            pl.BlockSpec((512, 256), lambda i: (0, 0)),
        ],
        out_specs=[pl.BlockSpec((B, 128), lambda i: (i, 0))] * 2,
        out_shape=[jax.ShapeDtypeStruct((n, 128), F32)] * 2,
    )(z1a, z1b, deg0, deg1, W1, b1.reshape(1, 512), W2)


def _tc_heads(z2a, z2b, deg0, deg1, b2, Wc1, bc1, Wc2, bc2, W3p, b3p,
              Wr1, br1, Wr2, br2, Wr3, br3, n):
    B = 2000

    def body(za_r, zb_r, d0_r, d1_r, b2_r, Wc1_r, bc1_r, Wc2_r, bc2_r,
             W3_r, b3_r, Wr1_r, br1_r, Wr2_r, br2_r, Wr3_r, br3_r,
             c_r, r_r, h_r):
        dv = _dinv_block(d0_r, d1_r)
        z = jnp.concatenate([za_r[...], zb_r[...]], axis=1) * dv
        h = jnp.tanh(z + b2_r[...])
        h_r[...] = h
        c1 = jnp.maximum(_dot16(h, Wc1_r[...])
                         + bc1_r[...], 0.0)
        c2 = jnp.maximum(_dot16(c1, Wc2_r[...])
                         + bc2_r[...], 0.0)
        logits = _dot16(c2, W3_r[...]) + b3_r[...]
        m = jnp.max(logits, axis=1, keepdims=True)
        ex = jnp.exp(logits - m)
        p = ex / jnp.sum(ex, axis=1, keepdims=True)
        c_r[...] = p[:, :16]
        r1 = jnp.maximum(_dot16(h, Wr1_r[...])
                         + br1_r[...], 0.0)
        r2 = jnp.maximum(_dot16(r1, Wr2_r[...])
                         + br2_r[...], 0.0)
        r_r[...] = _dot16(r2, Wr3_r[...]) + br3_r[...]

    full = lambda a, b: pl.BlockSpec((a, b), lambda i: (0, 0))
    return pl.pallas_call(
        body,
        grid=(n // B,),
        in_specs=[
            pl.BlockSpec((B, 128), lambda i: (i, 0)),
            pl.BlockSpec((B, 128), lambda i: (i, 0)),
            pl.BlockSpec((B, 128), lambda i: (i, 0)),
            pl.BlockSpec((B, 128), lambda i: (i, 0)),
            full(1, 256), full(256, 256), full(1, 256), full(256, 128),
            full(1, 128), full(128, 128), full(1, 128), full(256, 256),
            full(1, 256), full(256, 512), full(1, 512), full(512, 256),
            full(1, 256),
        ],
        out_specs=[
            pl.BlockSpec((B, 16), lambda i: (i, 0)),
            pl.BlockSpec((B, 256), lambda i: (i, 0)),
            pl.BlockSpec((B, 256), lambda i: (i, 0)),
        ],
        out_shape=[
            jax.ShapeDtypeStruct((n, 16), F32),
            jax.ShapeDtypeStruct((n, 256), F32),
            jax.ShapeDtypeStruct((n, 256), F32),
        ],
    )(z2a, z2b, deg0, deg1, b2, Wc1, bc1, Wc2, bc2, W3p, b3p,
      Wr1, br1, Wr2, br2, Wr3, br3)


def kernel(x, edge_index, W_gc1, b_gc1, W_gc2, b_gc2, W_c1, b_c1, g_c1, be_c1,
           W_c2, b_c2, g_c2, be_c2, W_c3, b_c3, W_r1, b_r1, W_r2, b_r2,
           W_r3, b_r3):
    n, d = x.shape
    e = edge_index.shape[1]
    src = edge_index[0]
    dst = edge_index[1]

    zeros_img = jnp.zeros((n, 128), F32)
    ones_img = jnp.ones((128, 128), F32)

    bn = np.float32(1.0 / np.sqrt(1.0 + 1e-5))
    Wc1 = W_c1 * (g_c1 * bn)[None, :]
    bc1 = b_c1 * (g_c1 * bn) + be_c1
    Wc2 = W_c2 * (g_c2 * bn)[None, :]
    bc2 = b_c2 * (g_c2 * bn) + be_c2
    W3p = jnp.pad(W_c3, ((0, 0), (0, 112)))
    b3p = jnp.pad(b_c3, (0, 112), constant_values=-1e30)

    deg0, deg1 = _deg_hist(dst, zeros_img, ones_img, n, e)
    y1a, y1b = _tc_prescale(x, deg0, deg1, n)
    z1a, z1b = _sc_aggregate(y1a, y1b, src, dst, n, e)
    y2a, y2b = _tc_conv_mid(z1a, z1b, deg0, deg1, W_gc1, b_gc1, W_gc2, n)
    z2a, z2b = _sc_aggregate(y2a, y2b, src, dst, n, e)
    c, r, h = _tc_heads(z2a, z2b, deg0, deg1, b_gc2.reshape(1, 256),
                        Wc1, bc1.reshape(1, 256), Wc2, bc2.reshape(1, 128),
                        W3p, b3p.reshape(1, 128),
                        W_r1, b_r1.reshape(1, 256), W_r2, b_r2.reshape(1, 512),
                        W_r3, b_r3.reshape(1, 256), n)
    return (c, r, h)


# speedup vs baseline: 1.0233x; 1.0051x over previous
import functools

import jax
import jax.numpy as jnp
import numpy as np
from jax import lax
from jax.experimental import pallas as pl
from jax.experimental.pallas import tpu as pltpu
from jax.experimental.pallas import tpu_sc as plsc

F32 = jnp.float32
I32 = jnp.int32


def _sc_mesh():
    return plsc.VectorSubcoreMesh(core_axis_name="c", subcore_axis_name="s",
                                  num_cores=2, num_subcores=16)


def _per_tile_row_copy(s, n, copy_fn):
    rpt = (n // 16) // 8 * 8
    last = n - 15 * rpt

    @pl.when(s < 15)
    def _():
        copy_fn(pl.multiple_of(s * rpt, 8), rpt)

    @pl.when(s == 15)
    def _():
        copy_fn(15 * rpt, last)


def _deg_hist(dst, zeros_img, ones_img, n, e):
    ch = 128
    e_core = e // 2
    e_sub = e_core // 16
    nch = e_sub // ch
    tail = e_sub - nch * ch

    @functools.partial(
        pl.kernel,
        out_type=(jax.ShapeDtypeStruct((n, 128), F32),
                  jax.ShapeDtypeStruct((n, 128), F32)),
        mesh=_sc_mesh(),
        scratch_types=[
            pltpu.VMEM_SHARED((n, 128), F32),
            pltpu.VMEM((ch,), I32), pltpu.VMEM((ch,), I32),
            pltpu.VMEM((ch,), I32), pltpu.VMEM((ch,), I32),
            pltpu.VMEM((ch, 128), F32),
            pltpu.VMEM((max(tail, 1),), I32),
            pltpu.SemaphoreType.DMA, pltpu.SemaphoreType.DMA,
            pltpu.SemaphoreType.DMA, pltpu.SemaphoreType.DMA,
        ],
    )
    def k(dst_h, zeros_h, ones_h, out0, out1, acc, di0, di1, di2, di3,
          ones_v, idxt_v, semI0, semI1, semI2, semI3):
        c = lax.axis_index("c")
        s = lax.axis_index("s")
        eb = c * e_core + s * e_sub
        islots = ((di0, semI0), (di1, semI1), (di2, semI2), (di3, semI3))

        def idx_load(j, sl):
            b = pl.multiple_of(eb + j * ch, 8)
            pltpu.async_copy(dst_h.at[pl.ds(b, ch)], sl[0], sl[1])

        def idx_wait(sl):
            pltpu.make_async_copy(dst_h.at[pl.ds(0, ch)], sl[0], sl[1]).wait()

        def scat(sl):
            pltpu.sync_copy(ones_v, acc.at[sl[0]], add=True)

        idx_load(0, islots[0])
        idx_load(1, islots[1])
        _per_tile_row_copy(s, n, lambda r0, nr: pltpu.sync_copy(
            zeros_h.at[pl.ds(r0, nr)], acc.at[pl.ds(r0, nr)]))
        pltpu.sync_copy(ones_h, ones_v)
        plsc.subcore_barrier()

        def segment(jb, sA, sB, sC, sD):
            @pl.when(jb + 2 < nch)
            def _():
                idx_load(jb + 2, sC)

            idx_wait(sA)
            scat(sA)

            @pl.when(jb + 3 < nch)
            def _():
                idx_load(jb + 3, sD)

            idx_wait(sB)
            scat(sB)

        def quad(m, carry):
            j0 = 4 * m
            segment(j0, islots[0], islots[1], islots[2], islots[3])
            segment(j0 + 2, islots[2], islots[3], islots[0], islots[1])
            return carry

        lax.fori_loop(0, nch // 4, quad, 0)
        rem = nch % 4
        jr = nch - rem
        if rem >= 2:
            segment(jr, islots[jr % 4], islots[(jr + 1) % 4],
                    islots[(jr + 2) % 4], islots[(jr + 3) % 4])
        if rem % 2:
            sl = islots[(nch - 1) % 4]
            idx_wait(sl)
            scat(sl)
        if tail:
            b = pl.multiple_of(eb + nch * ch, 8)
            pltpu.sync_copy(dst_h.at[pl.ds(b, tail)], idxt_v)
            pltpu.sync_copy(ones_v.at[pl.ds(0, tail)], acc.at[idxt_v], add=True)
        plsc.subcore_barrier()

        @pl.when(c == 0)
        def _():
            _per_tile_row_copy(s, n, lambda r0, nr: pltpu.sync_copy(
                acc.at[pl.ds(r0, nr)], out0.at[pl.ds(r0, nr)]))

        @pl.when(c == 1)
        def _():
            _per_tile_row_copy(s, n, lambda r0, nr: pltpu.sync_copy(
                acc.at[pl.ds(r0, nr)], out1.at[pl.ds(r0, nr)]))

    return k(dst, zeros_img, ones_img)


def _sc_aggregate(ya, yb, src, dst, n, e):
    ch = 128
    e_sub = e // 16
    nch = e_sub // ch
    tail = e_sub - nch * ch
    assert nch % 2 == 0

    @functools.partial(
        pl.kernel,
        out_type=(jax.ShapeDtypeStruct((n, 128), F32),
                  jax.ShapeDtypeStruct((n, 128), F32)),
        mesh=_sc_mesh(),
        scratch_types=[
            pltpu.VMEM_SHARED((n, 128), F32),
            pltpu.VMEM((ch,), I32), pltpu.VMEM((ch,), I32),
            pltpu.VMEM((ch,), I32), pltpu.VMEM((ch,), I32),
            pltpu.VMEM((ch,), I32), pltpu.VMEM((ch,), I32),
            pltpu.VMEM((ch,), I32), pltpu.VMEM((ch,), I32),
            pltpu.VMEM((ch, 128), F32),
            pltpu.VMEM((ch, 128), F32),
            pltpu.VMEM((max(tail, 1),), I32),
            pltpu.VMEM((max(tail, 1),), I32),
            pltpu.VMEM((max(tail, 1), 128), F32),
            pltpu.SemaphoreType.DMA, pltpu.SemaphoreType.DMA,
            pltpu.SemaphoreType.DMA, pltpu.SemaphoreType.DMA,
            pltpu.SemaphoreType.DMA, pltpu.SemaphoreType.DMA,
        ],
    )
    def k(ya_h, yb_h, src_h, dst_h, za_h, zb_h,
          acc, si0, si1, si2, si3, di0, di1, di2, di3, rows0, rows1,
          sit, dit, rowst, semA, semB, semI0, semI1, semI2, semI3):
        c = lax.axis_index("c")
        s = lax.axis_index("s")
        eb = s * e_sub

        def run(y_h, z_h):
            islots = ((si0, di0, semI0), (si1, di1, semI1),
                      (si2, di2, semI2), (si3, di3, semI3))

            def idx_load(j, sl):
                si, di, semI = sl
                b = pl.multiple_of(eb + j * ch, 8)
                pltpu.async_copy(src_h.at[pl.ds(b, ch)], si, semI)
                pltpu.async_copy(dst_h.at[pl.ds(b, ch)], di, semI)

            def idx_wait(sl):
                si, di, semI = sl
                pltpu.make_async_copy(src_h.at[pl.ds(0, ch)], si, semI).wait()
                pltpu.make_async_copy(dst_h.at[pl.ds(0, ch)], di, semI).wait()

            def gather(sl, buf, sem):
                pltpu.async_copy(y_h.at[sl[0]], buf, sem)

            def gather_wait(sl, buf, sem):
                pltpu.make_async_copy(y_h.at[sl[0]], buf, sem).wait()

            def scat(sl, buf):
                pltpu.sync_copy(buf, acc.at[sl[1]], add=True)

            idx_load(0, islots[0])
            idx_load(1, islots[1])
            _per_tile_row_copy(s, n, lambda r0, nr: pltpu.sync_copy(
                y_h.at[pl.ds(r0, nr)], acc.at[pl.ds(r0, nr)]))
            plsc.subcore_barrier()

            idx_wait(islots[0])
            gather(islots[0], rows0, semA)

            def segment(jb, sA, sB, sC, sD):
                idx_wait(sB)
                gather(sB, rows1, semB)
                gather_wait(sA, rows0, semA)

                @pl.when(jb + 2 < nch)
                def _():
                    idx_load(jb + 2, sC)

                scat(sA, rows0)

                @pl.when(jb + 2 < nch)
                def _():
                    idx_wait(sC)
                    gather(sC, rows0, semA)

                gather_wait(sB, rows1, semB)
                scat(sB, rows1)

                @pl.when(jb + 3 < nch)
                def _():
                    idx_load(jb + 3, sD)

            def quad(m, carry):
                j0 = 4 * m
                segment(j0, islots[0], islots[1], islots[2], islots[3])
                segment(j0 + 2, islots[2], islots[3], islots[0], islots[1])
                return carry

            lax.fori_loop(0, nch // 4, quad, 0)
            if nch % 4:
                assert nch % 4 == 2
                segment(nch - 2, islots[0], islots[1], islots[2], islots[3])
            if tail:
                b = pl.multiple_of(eb + nch * ch, 8)
                pltpu.sync_copy(src_h.at[pl.ds(b, tail)], sit)
                pltpu.sync_copy(dst_h.at[pl.ds(b, tail)], dit)
                pltpu.async_copy(y_h.at[sit], rowst, semA).wait()
                pltpu.sync_copy(rowst, acc.at[dit], add=True)
            plsc.subcore_barrier()
            _per_tile_row_copy(s, n, lambda r0, nr: pltpu.sync_copy(
                acc.at[pl.ds(r0, nr)], z_h.at[pl.ds(r0, nr)]))

        @pl.when(c == 0)
        def _():
            run(ya_h, za_h)

        @pl.when(c == 1)
        def _():
            run(yb_h, zb_h)

    return k(ya, yb, src, dst)


def _dinv_block(d0, d1):
    return lax.rsqrt(d0[:, 0:1] + d1[:, 0:1] + 1.0)


def _dot16(a, w):
    return jnp.dot(a.astype(jnp.bfloat16), w.astype(jnp.bfloat16),
                   preferred_element_type=F32)


def _tc_prescale(x, deg0, deg1, n):
    B = 5000

    def body(x_r, d0_r, d1_r, ya_r, yb_r):
        dv = _dinv_block(d0_r, d1_r)
        xv = x_r[...]
        ya_r[...] = xv[:, :128] * dv
        yb_r[...] = xv[:, 128:] * dv

    return pl.pallas_call(
        body,
        grid=(n // B,),
        in_specs=[
            pl.BlockSpec((B, 256), lambda i: (i, 0)),
            pl.BlockSpec((B, 128), lambda i: (i, 0)),
            pl.BlockSpec((B, 128), lambda i: (i, 0)),
        ],
        out_specs=[pl.BlockSpec((B, 128), lambda i: (i, 0))] * 2,
        out_shape=[jax.ShapeDtypeStruct((n, 128), F32)] * 2,
    )(x, deg0, deg1)


def _tc_conv_mid(z1a, z1b, deg0, deg1, W1, b1, W2, n):
    B = 5000

    def body(za_r, zb_r, d0_r, d1_r, W1_r, b1_r, W2_r, ya_r, yb_r):
        dv = _dinv_block(d0_r, d1_r)
        s1 = jnp.concatenate([za_r[...], zb_r[...]], axis=1) * dv
        h1 = jnp.tanh(_dot16(s1, W1_r[...])
                      + b1_r[...])
        t = _dot16(h1, W2_r[...]) * dv
        ya_r[...] = t[:, :128]
        yb_r[...] = t[:, 128:]

    return pl.pallas_call(
        body,
        grid=(n // B,),
        in_specs=[
            pl.BlockSpec((B, 128), lambda i: (i, 0)),
            pl.BlockSpec((B, 128), lambda i: (i, 0)),
            pl.BlockSpec((B, 128), lambda i: (i, 0)),
            pl.BlockSpec((B, 128), lambda i: (i, 0)),
            pl.BlockSpec((256, 512), lambda i: (0, 0)),
            pl.BlockSpec((1, 512), lambda i: (0, 0)),
            pl.BlockSpec((512, 256), lambda i: (0, 0)),
        ],
        out_specs=[pl.BlockSpec((B, 128), lambda i: (i, 0))] * 2,
        out_shape=[jax.ShapeDtypeStruct((n, 128), F32)] * 2,
    )(z1a, z1b, deg0, deg1, W1, b1.reshape(1, 512), W2)


def _tc_heads(z2a, z2b, deg0, deg1, b2, Wc1, bc1, Wc2, bc2, W3p, b3p,
              Wr1, br1, Wr2, br2, Wr3, br3, n):
    B = 2000

    def body(za_r, zb_r, d0_r, d1_r, b2_r, Wc1_r, bc1_r, Wc2_r, bc2_r,
             W3_r, b3_r, Wr1_r, br1_r, Wr2_r, br2_r, Wr3_r, br3_r,
             c_r, r_r, h_r):
        dv = _dinv_block(d0_r, d1_r)
        z = jnp.concatenate([za_r[...], zb_r[...]], axis=1) * dv
        h = jnp.tanh(z + b2_r[...])
        h_r[...] = h
        c1 = jnp.maximum(_dot16(h, Wc1_r[...])
                         + bc1_r[...], 0.0)
        c2 = jnp.maximum(_dot16(c1, Wc2_r[...])
                         + bc2_r[...], 0.0)
        logits = _dot16(c2, W3_r[...]) + b3_r[...]
        m = jnp.max(logits, axis=1, keepdims=True)
        ex = jnp.exp(logits - m)
        p = ex / jnp.sum(ex, axis=1, keepdims=True)
        c_r[...] = p[:, :16]
        r1 = jnp.maximum(_dot16(h, Wr1_r[...])
                         + br1_r[...], 0.0)
        r2 = jnp.maximum(_dot16(r1, Wr2_r[...])
                         + br2_r[...], 0.0)
        r_r[...] = _dot16(r2, Wr3_r[...]) + br3_r[...]

    full = lambda a, b: pl.BlockSpec((a, b), lambda i: (0, 0))
    return pl.pallas_call(
        body,
        grid=(n // B,),
        in_specs=[
            pl.BlockSpec((B, 128), lambda i: (i, 0)),
            pl.BlockSpec((B, 128), lambda i: (i, 0)),
            pl.BlockSpec((B, 128), lambda i: (i, 0)),
            pl.BlockSpec((B, 128), lambda i: (i, 0)),
            full(1, 256), full(256, 256), full(1, 256), full(256, 128),
            full(1, 128), full(128, 128), full(1, 128), full(256, 256),
            full(1, 256), full(256, 512), full(1, 512), full(512, 256),
            full(1, 256),
        ],
        out_specs=[
            pl.BlockSpec((B, 16), lambda i: (i, 0)),
            pl.BlockSpec((B, 256), lambda i: (i, 0)),
            pl.BlockSpec((B, 256), lambda i: (i, 0)),
        ],
        out_shape=[
            jax.ShapeDtypeStruct((n, 16), F32),
            jax.ShapeDtypeStruct((n, 256), F32),
            jax.ShapeDtypeStruct((n, 256), F32),
        ],
    )(z2a, z2b, deg0, deg1, b2, Wc1, bc1, Wc2, bc2, W3p, b3p,
      Wr1, br1, Wr2, br2, Wr3, br3)


def kernel(x, edge_index, W_gc1, b_gc1, W_gc2, b_gc2, W_c1, b_c1, g_c1, be_c1,
           W_c2, b_c2, g_c2, be_c2, W_c3, b_c3, W_r1, b_r1, W_r2, b_r2,
           W_r3, b_r3):
    n, d = x.shape
    e = edge_index.shape[1]
    src = edge_index[0]
    dst = edge_index[1]

    zeros_img = jnp.zeros((n, 128), F32)
    ones_img = jnp.ones((128, 128), F32)

    bn = np.float32(1.0 / np.sqrt(1.0 + 1e-5))
    Wc1 = W_c1 * (g_c1 * bn)[None, :]
    bc1 = b_c1 * (g_c1 * bn) + be_c1
    Wc2 = W_c2 * (g_c2 * bn)[None, :]
    bc2 = b_c2 * (g_c2 * bn) + be_c2
    W3p = jnp.pad(W_c3, ((0, 0), (0, 112)))
    b3p = jnp.pad(b_c3, (0, 112), constant_values=-1e30)

    deg0, deg1 = _deg_hist(dst, zeros_img, ones_img, n, e)
    y1a, y1b = _tc_prescale(x, deg0, deg1, n)
    z1a, z1b = _sc_aggregate(y1a, y1b, src, dst, n, e)
    y2a, y2b = _tc_conv_mid(z1a, z1b, deg0, deg1, W_gc1, b_gc1, W_gc2, n)
    z2a, z2b = _sc_aggregate(y2a, y2b, src, dst, n, e)
    c, r, h = _tc_heads(z2a, z2b, deg0, deg1, b_gc2.reshape(1, 256),
                        Wc1, bc1.reshape(1, 256), Wc2, bc2.reshape(1, 128),
                        W3p, b3p.reshape(1, 128),
                        W_r1, b_r1.reshape(1, 256), W_r2, b_r2.reshape(1, 512),
                        W_r3, b_r3.reshape(1, 256), n)
    return (c, r, h)
